# baseline (speedup 1.0000x reference)
import functools

import jax
import jax.numpy as jnp
from jax import lax
from jax.experimental import pallas as pl
from jax.experimental.pallas import tpu as pltpu
from jax.experimental.pallas import tpu_sc as plsc

L = 16
NC, NS = 2, 16
NW = NC * NS
C = 4096
ROWS = 4 * 4096
ROWS_PER_W = ROWS // NW
CHUNK = 8
NCHUNK = ROWS_PER_W // CHUNK

_mesh = plsc.VectorSubcoreMesh(
    core_axis_name="c", subcore_axis_name="s", num_cores=NC, num_subcores=NS
)


@functools.partial(
    pl.kernel,
    out_type=jax.ShapeDtypeStruct((ROWS, C), jnp.float32),
    mesh=_mesh,
    scratch_types=[
        pltpu.VMEM((C,), jnp.int32),
        pltpu.VMEM((CHUNK, C), jnp.float32),
        pltpu.VMEM((CHUNK, C), jnp.float32),
    ],
    compiler_params=pltpu.CompilerParams(needs_layout_passes=False),
)
def _shuffle_sc(x_hbm, idx_hbm, out_hbm, idx_v, in_v, out_v):
    wid = lax.axis_index("s") * NC + lax.axis_index("c")
    base = wid * ROWS_PER_W

    pltpu.sync_copy(idx_hbm, idx_v)

    def chunk_body(g, _):
        row0 = base + g * CHUNK
        pltpu.sync_copy(x_hbm.at[pl.ds(row0, CHUNK), :], in_v)

        def vec_body(j, _):
            col = j * L
            idxv = idx_v[pl.ds(col, L)]
            for r in range(CHUNK):
                rvec = jnp.full((L,), r, jnp.int32)
                out_v[r, pl.ds(col, L)] = plsc.load_gather(in_v, [rvec, idxv])
            return 0

        lax.fori_loop(0, C // L, vec_body, 0, unroll=2)
        pltpu.sync_copy(out_v, out_hbm.at[pl.ds(row0, CHUNK), :])
        return 0

    lax.fori_loop(0, NCHUNK, chunk_body, 0)


def kernel(x, forward_shuffle_idx):
    x2 = x.reshape(ROWS, C)
    out = _shuffle_sc(x2, forward_shuffle_idx)
    return out.reshape(x.shape)


# speedup vs baseline: 1.1296x; 1.1296x over previous
import functools

import jax
import jax.numpy as jnp
from jax import lax
from jax.experimental import pallas as pl
from jax.experimental.pallas import tpu as pltpu
from jax.experimental.pallas import tpu_sc as plsc

L = 16
NC, NS = 2, 16
NW = NC * NS
C = 4096
ROWS = 4 * 4096
ROWS_PER_W = ROWS // NW
CHUNK = 4
NCHUNK = ROWS_PER_W // CHUNK
NPAIR = NCHUNK // 2

_mesh = plsc.VectorSubcoreMesh(
    core_axis_name="c", subcore_axis_name="s", num_cores=NC, num_subcores=NS
)


@functools.partial(
    pl.kernel,
    out_type=jax.ShapeDtypeStruct((ROWS, C), jnp.float32),
    mesh=_mesh,
    scratch_types=[
        pltpu.VMEM((C,), jnp.int32),
        pltpu.VMEM((CHUNK, C), jnp.float32),
        pltpu.VMEM((CHUNK, C), jnp.float32),
        pltpu.VMEM((CHUNK, C), jnp.float32),
        pltpu.VMEM((CHUNK, C), jnp.float32),
        pltpu.SemaphoreType.DMA,
        pltpu.SemaphoreType.DMA,
        pltpu.SemaphoreType.DMA,
        pltpu.SemaphoreType.DMA,
    ],
    compiler_params=pltpu.CompilerParams(needs_layout_passes=False),
)
def _shuffle_sc(x_hbm, idx_hbm, out_hbm, idx_v, in0, in1, out0, out1,
                isem0, isem1, osem0, osem1):
    wid = lax.axis_index("s") * NC + lax.axis_index("c")
    base = wid * ROWS_PER_W

    ins = (in0, in1)
    outs = (out0, out1)
    isems = (isem0, isem1)
    osems = (osem0, osem1)

    pltpu.sync_copy(idx_hbm, idx_v)

    def start_in(g, b):
        row0 = base + g * CHUNK
        pltpu.async_copy(x_hbm.at[pl.ds(row0, CHUNK), :], ins[b], isems[b])

    def wait_in(g, b):
        row0 = base + g * CHUNK
        pltpu.make_async_copy(
            x_hbm.at[pl.ds(row0, CHUNK), :], ins[b], isems[b]
        ).wait()

    def start_out(g, b):
        row0 = base + g * CHUNK
        pltpu.async_copy(outs[b], out_hbm.at[pl.ds(row0, CHUNK), :], osems[b])

    def wait_out(g, b):
        row0 = base + g * CHUNK
        pltpu.make_async_copy(
            outs[b], out_hbm.at[pl.ds(row0, CHUNK), :], osems[b]
        ).wait()

    def compute(b):
        in_v = ins[b]
        out_v = outs[b]

        def vec_body(j, _):
            col = j * L
            idxv = idx_v[pl.ds(col, L)]
            for r in range(CHUNK):
                rvec = jnp.full((L,), r, jnp.int32)
                out_v[r, pl.ds(col, L)] = plsc.load_gather(in_v, [rvec, idxv])
            return 0

        lax.fori_loop(0, C // L, vec_body, 0, unroll=2)

    start_in(0, 0)
    start_in(1, 1)

    def pair_body(p, _):
        for b in range(2):
            g = 2 * p + b

            @pl.when(p > 0)
            def _():
                wait_out(g - 2, b)

            wait_in(g, b)
            compute(b)
            start_out(g, b)

            @pl.when(p < NPAIR - 1)
            def _():
                start_in(g + 2, b)

        return 0

    lax.fori_loop(0, NPAIR, pair_body, 0)
    wait_out(NCHUNK - 2, 0)
    wait_out(NCHUNK - 1, 1)


def kernel(x, forward_shuffle_idx):
    x2 = x.reshape(ROWS, C)
    out = _shuffle_sc(x2, forward_shuffle_idx)
    return out.reshape(x.shape)


# speedup vs baseline: 4.2994x; 3.8062x over previous
import functools

import jax
import jax.numpy as jnp
from jax import lax
from jax.experimental import pallas as pl
from jax.experimental.pallas import tpu as pltpu
from jax.experimental.pallas import tpu_sc as plsc

L = 16
NC, NS = 2, 16
NW = NC * NS
C = 4096
ROWS = 4 * 4096
ROWS_PER_W = ROWS // NW
CHUNK = 4
NCHUNK = ROWS_PER_W // CHUNK
NPAIR = NCHUNK // 2

_mesh = plsc.VectorSubcoreMesh(
    core_axis_name="c", subcore_axis_name="s", num_cores=NC, num_subcores=NS
)


@functools.partial(
    pl.kernel,
    out_type=jax.ShapeDtypeStruct((ROWS, C), jnp.float32),
    mesh=_mesh,
    scratch_types=[
        pltpu.VMEM((C,), jnp.int32),
        pltpu.VMEM((CHUNK, C), jnp.float32),
        pltpu.VMEM((CHUNK, C), jnp.float32),
        pltpu.VMEM((CHUNK, C), jnp.float32),
        pltpu.VMEM((CHUNK, C), jnp.float32),
        pltpu.SemaphoreType.DMA,
        pltpu.SemaphoreType.DMA,
        pltpu.SemaphoreType.DMA,
        pltpu.SemaphoreType.DMA,
    ],
    compiler_params=pltpu.CompilerParams(needs_layout_passes=False),
)
def _shuffle_sc(x_hbm, idx_hbm, out_hbm, idx_v, in0, in1, out0, out1,
                isem0, isem1, osem0, osem1):
    wid = lax.axis_index("s") * NC + lax.axis_index("c")
    base = wid * ROWS_PER_W

    ins = (in0, in1)
    outs = (out0, out1)
    isems = (isem0, isem1)
    osems = (osem0, osem1)

    pltpu.sync_copy(idx_hbm, idx_v)

    def start_in(g, b):
        row0 = base + g * CHUNK
        pltpu.async_copy(x_hbm.at[pl.ds(row0, CHUNK), :], ins[b], isems[b])

    def wait_in(g, b):
        row0 = base + g * CHUNK
        pltpu.make_async_copy(
            x_hbm.at[pl.ds(row0, CHUNK), :], ins[b], isems[b]
        ).wait()

    def start_out(g, b):
        row0 = base + g * CHUNK
        pltpu.async_copy(outs[b], out_hbm.at[pl.ds(row0, CHUNK), :], osems[b])

    def wait_out(g, b):
        row0 = base + g * CHUNK
        pltpu.make_async_copy(
            outs[b], out_hbm.at[pl.ds(row0, CHUNK), :], osems[b]
        ).wait()

    def compute(b):
        in_v = ins[b]
        out_v = outs[b]

        @plsc.parallel_loop(0, C // L, unroll=4)
        def _(j):
            col = j * L
            idxv = idx_v[pl.ds(col, L)]
            for r in range(CHUNK):
                rvec = jnp.full((L,), r, jnp.int32)
                out_v[r, pl.ds(col, L)] = plsc.load_gather(in_v, [rvec, idxv])

    start_in(0, 0)
    start_in(1, 1)

    def pair_body(p, _):
        for b in range(2):
            g = 2 * p + b

            @pl.when(p > 0)
            def _():
                wait_out(g - 2, b)

            wait_in(g, b)
            compute(b)
            start_out(g, b)

            @pl.when(p < NPAIR - 1)
            def _():
                start_in(g + 2, b)

        return 0

    lax.fori_loop(0, NPAIR, pair_body, 0)
    wait_out(NCHUNK - 2, 0)
    wait_out(NCHUNK - 1, 1)


def kernel(x, forward_shuffle_idx):
    x2 = x.reshape(ROWS, C)
    out = _shuffle_sc(x2, forward_shuffle_idx)
    return out.reshape(x.shape)


# speedup vs baseline: 4.3112x; 1.0028x over previous
import functools

import jax
import jax.numpy as jnp
from jax import lax
from jax.experimental import pallas as pl
from jax.experimental.pallas import tpu as pltpu
from jax.experimental.pallas import tpu_sc as plsc

L = 16
NC, NS = 2, 16
NW = NC * NS
C = 4096
ROWS = 4 * 4096
ROWS_PER_W = ROWS // NW
CHUNK = 4
NCHUNK = ROWS_PER_W // CHUNK
NPAIR = NCHUNK // 2

_mesh = plsc.VectorSubcoreMesh(
    core_axis_name="c", subcore_axis_name="s", num_cores=NC, num_subcores=NS
)


@functools.partial(
    pl.kernel,
    out_type=jax.ShapeDtypeStruct((ROWS, C), jnp.float32),
    mesh=_mesh,
    scratch_types=[
        pltpu.VMEM((C,), jnp.int32),
        pltpu.VMEM((CHUNK, C), jnp.float32),
        pltpu.VMEM((CHUNK, C), jnp.float32),
        pltpu.VMEM((CHUNK, C), jnp.float32),
        pltpu.VMEM((CHUNK, C), jnp.float32),
        pltpu.SemaphoreType.DMA,
        pltpu.SemaphoreType.DMA,
        pltpu.SemaphoreType.DMA,
        pltpu.SemaphoreType.DMA,
    ],
    compiler_params=pltpu.CompilerParams(needs_layout_passes=False),
)
def _shuffle_sc(x_hbm, idx_hbm, out_hbm, idx_v, in0, in1, out0, out1,
                isem0, isem1, osem0, osem1):
    wid = lax.axis_index("s") * NC + lax.axis_index("c")
    base = wid * ROWS_PER_W

    ins = (in0, in1)
    outs = (out0, out1)
    isems = (isem0, isem1)
    osems = (osem0, osem1)

    pltpu.sync_copy(idx_hbm, idx_v)

    def start_in(g, b):
        row0 = base + g * CHUNK
        pltpu.async_copy(x_hbm.at[pl.ds(row0, CHUNK), :], ins[b], isems[b])

    def wait_in(g, b):
        row0 = base + g * CHUNK
        pltpu.make_async_copy(
            x_hbm.at[pl.ds(row0, CHUNK), :], ins[b], isems[b]
        ).wait()

    def start_out(g, b):
        row0 = base + g * CHUNK
        pltpu.async_copy(outs[b], out_hbm.at[pl.ds(row0, CHUNK), :], osems[b])

    def wait_out(g, b):
        row0 = base + g * CHUNK
        pltpu.make_async_copy(
            outs[b], out_hbm.at[pl.ds(row0, CHUNK), :], osems[b]
        ).wait()

    def compute(b):
        in_v = ins[b]
        out_v = outs[b]

        @plsc.parallel_loop(0, C // L, unroll=8)
        def _(j):
            col = j * L
            idxv = idx_v[pl.ds(col, L)]
            for r in range(CHUNK):
                rvec = jnp.full((L,), r, jnp.int32)
                out_v[r, pl.ds(col, L)] = plsc.load_gather(in_v, [rvec, idxv])

    start_in(0, 0)
    start_in(1, 1)

    def pair_body(p, _):
        for b in range(2):
            g = 2 * p + b

            @pl.when(p > 0)
            def _():
                wait_out(g - 2, b)

            wait_in(g, b)
            compute(b)
            start_out(g, b)

            @pl.when(p < NPAIR - 1)
            def _():
                start_in(g + 2, b)

        return 0

    lax.fori_loop(0, NPAIR, pair_body, 0)
    wait_out(NCHUNK - 2, 0)
    wait_out(NCHUNK - 1, 1)


def kernel(x, forward_shuffle_idx):
    x2 = x.reshape(ROWS, C)
    out = _shuffle_sc(x2, forward_shuffle_idx)
    return out.reshape(x.shape)


# speedup vs baseline: 4.4286x; 1.0272x over previous
import functools

import jax
import jax.numpy as jnp
from jax import lax
from jax.experimental import pallas as pl
from jax.experimental.pallas import tpu as pltpu
from jax.experimental.pallas import tpu_sc as plsc

L = 16
NC, NS = 2, 16
NW = NC * NS
C = 4096
ROWS = 4 * 4096
ROWS_PER_W = ROWS // NW
CIN = 8
COUT = 4
NIN = ROWS_PER_W // CIN
NOUT = ROWS_PER_W // COUT
NPAIR = NIN // 2

_mesh = plsc.VectorSubcoreMesh(
    core_axis_name="c", subcore_axis_name="s", num_cores=NC, num_subcores=NS
)


@functools.partial(
    pl.kernel,
    out_type=jax.ShapeDtypeStruct((ROWS, C), jnp.float32),
    mesh=_mesh,
    scratch_types=[
        pltpu.VMEM((C,), jnp.int32),
        pltpu.VMEM((CIN, C), jnp.float32),
        pltpu.VMEM((CIN, C), jnp.float32),
        pltpu.VMEM((COUT, C), jnp.float32),
        pltpu.VMEM((COUT, C), jnp.float32),
        pltpu.SemaphoreType.DMA,
        pltpu.SemaphoreType.DMA,
        pltpu.SemaphoreType.DMA,
        pltpu.SemaphoreType.DMA,
    ],
    compiler_params=pltpu.CompilerParams(needs_layout_passes=False),
)
def _shuffle_sc(x_hbm, idx_hbm, out_hbm, idx_v, in0, in1, out0, out1,
                isem0, isem1, osem0, osem1):
    wid = lax.axis_index("s") * NC + lax.axis_index("c")
    base = wid * ROWS_PER_W

    ins = (in0, in1)
    outs = (out0, out1)
    isems = (isem0, isem1)
    osems = (osem0, osem1)

    pltpu.sync_copy(idx_hbm, idx_v)

    def start_in(g, b):
        row0 = base + g * CIN
        pltpu.async_copy(x_hbm.at[pl.ds(row0, CIN), :], ins[b], isems[b])

    def wait_in(g, b):
        row0 = base + g * CIN
        pltpu.make_async_copy(
            x_hbm.at[pl.ds(row0, CIN), :], ins[b], isems[b]
        ).wait()

    def start_out(q, h):
        row0 = base + q * COUT
        pltpu.async_copy(outs[h], out_hbm.at[pl.ds(row0, COUT), :], osems[h])

    def wait_out(q, h):
        row0 = base + q * COUT
        pltpu.make_async_copy(
            outs[h], out_hbm.at[pl.ds(row0, COUT), :], osems[h]
        ).wait()

    def compute(b, h):
        in_v = ins[b]
        out_v = outs[h]

        @plsc.parallel_loop(0, C // L, unroll=8)
        def _(j):
            col = j * L
            idxv = idx_v[pl.ds(col, L)]
            for r in range(COUT):
                rvec = jnp.full((L,), COUT * h + r, jnp.int32)
                out_v[r, pl.ds(col, L)] = plsc.load_gather(in_v, [rvec, idxv])

    start_in(0, 0)
    start_in(1, 1)

    def body(g, b):
        wait_in(g, b)
        for h in range(2):
            q = 2 * g + h

            @pl.when(q >= 2)
            def _():
                wait_out(q - 2, h)

            compute(b, h)
            start_out(q, h)

        @pl.when(g + 2 < NIN)
        def _():
            start_in(g + 2, b)

    def pair_body(p, _):
        body(2 * p, 0)
        body(2 * p + 1, 1)
        return 0

    lax.fori_loop(0, NPAIR, pair_body, 0)
    wait_out(NOUT - 2, 0)
    wait_out(NOUT - 1, 1)


def kernel(x, forward_shuffle_idx):
    x2 = x.reshape(ROWS, C)
    out = _shuffle_sc(x2, forward_shuffle_idx)
    return out.reshape(x.shape)


# speedup vs baseline: 4.4411x; 1.0028x over previous
import functools

import jax
import jax.numpy as jnp
from jax import lax
from jax.experimental import pallas as pl
from jax.experimental.pallas import tpu as pltpu
from jax.experimental.pallas import tpu_sc as plsc

L = 16
NC, NS = 2, 16
NW = NC * NS
C = 4096
ROWS = 4 * 4096
ROWS_PER_W = ROWS // NW
CHUNK = 4
NCHUNK = ROWS_PER_W // CHUNK
NB = 3
NGROUP = (NCHUNK + NB - 1) // NB

_mesh = plsc.VectorSubcoreMesh(
    core_axis_name="c", subcore_axis_name="s", num_cores=NC, num_subcores=NS
)


@functools.partial(
    pl.kernel,
    out_type=jax.ShapeDtypeStruct((ROWS, C), jnp.float32),
    mesh=_mesh,
    scratch_types=[
        pltpu.VMEM((C,), jnp.int32),
        pltpu.VMEM((CHUNK, C), jnp.float32),
        pltpu.VMEM((CHUNK, C), jnp.float32),
        pltpu.VMEM((CHUNK, C), jnp.float32),
        pltpu.VMEM((CHUNK, C), jnp.float32),
        pltpu.VMEM((CHUNK, C), jnp.float32),
        pltpu.VMEM((CHUNK, C), jnp.float32),
        pltpu.SemaphoreType.DMA,
        pltpu.SemaphoreType.DMA,
        pltpu.SemaphoreType.DMA,
        pltpu.SemaphoreType.DMA,
        pltpu.SemaphoreType.DMA,
        pltpu.SemaphoreType.DMA,
    ],
    compiler_params=pltpu.CompilerParams(needs_layout_passes=False),
)
def _shuffle_sc(x_hbm, idx_hbm, out_hbm, idx_v, in0, in1, in2,
                out0, out1, out2, isem0, isem1, isem2, osem0, osem1, osem2):
    wid = lax.axis_index("s") * NC + lax.axis_index("c")
    base = wid * ROWS_PER_W

    ins = (in0, in1, in2)
    outs = (out0, out1, out2)
    isems = (isem0, isem1, isem2)
    osems = (osem0, osem1, osem2)

    pltpu.sync_copy(idx_hbm, idx_v)

    def start_in(g, b):
        row0 = base + g * CHUNK
        pltpu.async_copy(x_hbm.at[pl.ds(row0, CHUNK), :], ins[b], isems[b])

    def wait_in(g, b):
        row0 = base + g * CHUNK
        pltpu.make_async_copy(
            x_hbm.at[pl.ds(row0, CHUNK), :], ins[b], isems[b]
        ).wait()

    def start_out(g, b):
        row0 = base + g * CHUNK
        pltpu.async_copy(outs[b], out_hbm.at[pl.ds(row0, CHUNK), :], osems[b])

    def wait_out(g, b):
        row0 = base + g * CHUNK
        pltpu.make_async_copy(
            outs[b], out_hbm.at[pl.ds(row0, CHUNK), :], osems[b]
        ).wait()

    def compute(b):
        in_v = ins[b]
        out_v = outs[b]

        @plsc.parallel_loop(0, C // L, unroll=8)
        def _(j):
            col = j * L
            idxv = idx_v[pl.ds(col, L)]
            for r in range(CHUNK):
                rvec = jnp.full((L,), r, jnp.int32)
                out_v[r, pl.ds(col, L)] = plsc.load_gather(in_v, [rvec, idxv])

    for b in range(NB):
        start_in(b, b)

    def group_body(t, _):
        for b in range(NB):
            g = NB * t + b

            @pl.when(g < NCHUNK)
            def _():
                @pl.when(g >= NB)
                def _():
                    wait_out(g - NB, b)

                wait_in(g, b)
                compute(b)
                start_out(g, b)

                @pl.when(g + NB < NCHUNK)
                def _():
                    start_in(g + NB, b)

        return 0

    lax.fori_loop(0, NGROUP, group_body, 0)
    for i in range(NB):
        g = NCHUNK - NB + i
        wait_out(g, g % NB)


def kernel(x, forward_shuffle_idx):
    x2 = x.reshape(ROWS, C)
    out = _shuffle_sc(x2, forward_shuffle_idx)
    return out.reshape(x.shape)
